# Initial kernel scaffold; baseline (speedup 1.0000x reference)
import jax
import jax.numpy as jnp
from jax.experimental import pallas as pl


def kernel(x, edge_index, W1, b1, W2, b2, Wc, bc, Wf, bf):
    raise NotImplementedError("write your pallas kernel here")


# speedup vs baseline: 5.4555x; 5.4555x over previous
import functools

import jax
import jax.numpy as jnp
from jax import lax
from jax.experimental import pallas as pl
from jax.experimental.pallas import tpu as pltpu
from jax.experimental.pallas import tpu_sc as plsc

N = 10000
E = 320000
D = 128
NC = 2
NS = 16
NW = NC * NS
CHUNK = 128
NCHUNKS = E // CHUNK
NPAD = 10240
ROWS_PER_TILE = NPAD // NS

_MESH = plsc.VectorSubcoreMesh(core_axis_name="c", subcore_axis_name="s",
                               num_cores=NC, num_subcores=NS)


@functools.partial(
    pl.kernel,
    out_type=(jax.ShapeDtypeStruct((NPAD,), jnp.float32),
              jax.ShapeDtypeStruct((NPAD,), jnp.float32)),
    mesh=_MESH,
    scratch_types=[
        pltpu.VMEM((CHUNK,), jnp.int32),
        pltpu.VMEM((CHUNK,), jnp.float32),
        pltpu.VMEM_SHARED((NPAD,), jnp.float32),
    ],
)
def _degrees_sc(src_hbm, dst_hbm, zeros_hbm, outs_hbm, outd_hbm,
                idx_v, ones_v, acc_sh):
    c = lax.axis_index("c")
    s = lax.axis_index("s")
    seg = ROWS_PER_TILE

    for j in range(CHUNK // 16):
        ones_v[pl.ds(j * 16, 16)] = jnp.ones((16,), jnp.float32)

    pltpu.sync_copy(zeros_hbm.at[pl.ds(s * seg, seg)],
                    acc_sh.at[pl.ds(s * seg, seg)])
    plsc.subcore_barrier()

    nk_base = NCHUNKS // NS
    rem = NCHUNKS - nk_base * NS
    nk = nk_base + jnp.where(s < rem, 1, 0)

    def count_into(e_hbm):
        def body(i, carry):
            k = s + i * NS
            pltpu.sync_copy(e_hbm.at[pl.ds(k * CHUNK, CHUNK)], idx_v)
            pltpu.sync_copy(ones_v, acc_sh.at[idx_v], add=True)
            return carry
        lax.fori_loop(0, nk, body, 0)

    @pl.when(c == 0)
    def _():
        count_into(src_hbm)

    @pl.when(c == 1)
    def _():
        count_into(dst_hbm)

    plsc.subcore_barrier()

    @pl.when(c == 0)
    def _():
        pltpu.sync_copy(acc_sh.at[pl.ds(s * seg, seg)],
                        outs_hbm.at[pl.ds(s * seg, seg)])

    @pl.when(c == 1)
    def _():
        pltpu.sync_copy(acc_sh.at[pl.ds(s * seg, seg)],
                        outd_hbm.at[pl.ds(s * seg, seg)])


@functools.partial(
    pl.kernel,
    out_type=jax.ShapeDtypeStruct((NC, NPAD, D), jnp.float32),
    mesh=_MESH,
    scratch_types=[
        pltpu.VMEM((CHUNK,), jnp.int32),
        pltpu.VMEM((CHUNK,), jnp.int32),
        pltpu.VMEM((CHUNK, D), jnp.float32),
        pltpu.VMEM_SHARED((NPAD, D), jnp.float32),
        pltpu.SemaphoreType.DMA,
    ],
)
def _edge_agg_sc(m_hbm, src_hbm, dst_hbm, zrows_hbm, out_hbm,
                 src_v, dst_v, rows_v, acc_sh, sem):
    c = lax.axis_index("c")
    s = lax.axis_index("s")
    wid = s * NC + c

    pltpu.sync_copy(zrows_hbm.at[pl.ds(s * ROWS_PER_TILE, ROWS_PER_TILE)],
                    acc_sh.at[pl.ds(s * ROWS_PER_TILE, ROWS_PER_TILE)])
    plsc.subcore_barrier()

    nk_base = NCHUNKS // NW
    rem = NCHUNKS - nk_base * NW
    nk = nk_base + jnp.where(wid < rem, 1, 0)

    def body(i, carry):
        k = wid + i * NW
        pltpu.sync_copy(src_hbm.at[pl.ds(k * CHUNK, CHUNK)], src_v)
        pltpu.sync_copy(dst_hbm.at[pl.ds(k * CHUNK, CHUNK)], dst_v)
        pltpu.async_copy(m_hbm.at[src_v], rows_v, sem).wait()
        pltpu.sync_copy(rows_v, acc_sh.at[dst_v], add=True)
        return carry

    lax.fori_loop(0, nk, body, 0)
    plsc.subcore_barrier()

    pltpu.sync_copy(acc_sh.at[pl.ds(s * ROWS_PER_TILE, ROWS_PER_TILE)],
                    out_hbm.at[c, pl.ds(s * ROWS_PER_TILE, ROWS_PER_TILE)])


def _tc1_body(cnt_ref, x_ref, w1_ref, m1_ref, sout_ref, sin_ref):
    cnt = cnt_ref[...]
    sc = lax.rsqrt(jnp.maximum(cnt, 1.0))
    sout = sc[:, 0:1]
    sin = sc[:, 1:2]
    sout_ref[...] = sout
    sin_ref[...] = sin
    m1_ref[...] = jnp.dot(x_ref[...] * sout, w1_ref[...],
                          preferred_element_type=jnp.float32)


def _tc2_body(p_ref, sin_ref, b1_ref, sout_ref, w2_ref, m2_ref):
    agg = p_ref[0] + p_ref[1]
    h1 = jnp.maximum(agg * sin_ref[...] + b1_ref[...][None, :], 0.0)
    m2_ref[...] = jnp.dot(h1 * sout_ref[...], w2_ref[...],
                          preferred_element_type=jnp.float32)


def _tc3_body(p_ref, sin_ref, b2_ref, wc_ref, bc_ref, wf_ref, bf_ref,
              cat_ref, feat_ref):
    agg = p_ref[0] + p_ref[1]
    h2 = jnp.maximum(agg * sin_ref[...] + b2_ref[...][None, :], 0.0)
    cat_ref[...] = jnp.dot(h2, wc_ref[...],
                           preferred_element_type=jnp.float32) + bc_ref[...][None, :]
    feat_ref[...] = jnp.dot(h2, wf_ref[...],
                            preferred_element_type=jnp.float32) + bf_ref[...][None, :]


def kernel(x, edge_index, W1, b1, W2, b2, Wc, bc, Wf, bf):
    src = edge_index[0]
    dst = edge_index[1]
    zeros_cnt = jnp.zeros((NPAD,), jnp.float32)
    zeros_rows = jnp.zeros((NPAD, D), jnp.float32)

    cnt_src, cnt_dst = _degrees_sc(src, dst, zeros_cnt)
    cnt_t = jnp.stack([cnt_src[:N], cnt_dst[:N]], axis=1)

    m1, s_out, s_in = pl.pallas_call(
        _tc1_body,
        out_shape=(jax.ShapeDtypeStruct((N, D), jnp.float32),
                   jax.ShapeDtypeStruct((N, 1), jnp.float32),
                   jax.ShapeDtypeStruct((N, 1), jnp.float32)),
    )(cnt_t, x, W1)

    p1 = _edge_agg_sc(m1, src, dst, zeros_rows)

    m2 = pl.pallas_call(
        _tc2_body,
        out_shape=jax.ShapeDtypeStruct((N, D), jnp.float32),
    )(p1[:, :N], s_in, b1, s_out, W2)

    p2 = _edge_agg_sc(m2, src, dst, zeros_rows)

    cat, feat = pl.pallas_call(
        _tc3_body,
        out_shape=(jax.ShapeDtypeStruct((N, Wc.shape[1]), jnp.float32),
                   jax.ShapeDtypeStruct((N, D), jnp.float32)),
    )(p2[:, :N], s_in, b2, Wc, bc, Wf, bf)
    return (cat, feat)


# speedup vs baseline: 9.6545x; 1.7697x over previous
import functools

import jax
import jax.numpy as jnp
from jax import lax
from jax.experimental import pallas as pl
from jax.experimental.pallas import tpu as pltpu
from jax.experimental.pallas import tpu_sc as plsc

N = 10000
E = 320000
D = 128
NC = 2
NS = 16
NW = NC * NS
CHUNK = 128
NPAD = 10240
ROWS_PER_TILE = NPAD // NS
NCHUNKS = 2560
EPAD = NCHUNKS * CHUNK
K_AGG = NCHUNKS // NW
K_DEG = NCHUNKS // NS

_MESH = plsc.VectorSubcoreMesh(core_axis_name="c", subcore_axis_name="s",
                               num_cores=NC, num_subcores=NS)


@functools.partial(
    pl.kernel,
    out_type=(jax.ShapeDtypeStruct((NPAD,), jnp.float32),
              jax.ShapeDtypeStruct((NPAD,), jnp.float32)),
    mesh=_MESH,
    scratch_types=[
        pltpu.VMEM((CHUNK,), jnp.int32),
        pltpu.VMEM((CHUNK,), jnp.int32),
        pltpu.VMEM((CHUNK,), jnp.float32),
        pltpu.VMEM_SHARED((NPAD,), jnp.float32),
        pltpu.SemaphoreType.DMA,
        pltpu.SemaphoreType.DMA,
    ],
)
def _degrees_sc(src_hbm, dst_hbm, zeros_hbm, outs_hbm, outd_hbm,
                ib0, ib1, ones_v, acc_sh, si0, si1):
    c = lax.axis_index("c")
    s = lax.axis_index("s")
    seg = NPAD // NS

    for j in range(CHUNK // 16):
        ones_v[pl.ds(j * 16, 16)] = jnp.ones((16,), jnp.float32)

    pltpu.sync_copy(zeros_hbm.at[pl.ds(s * seg, seg)],
                    acc_sh.at[pl.ds(s * seg, seg)])
    plsc.subcore_barrier()

    def make_loop(e_hbm):
        def istart(j, ib, si):
            pltpu.async_copy(e_hbm.at[pl.ds((s * K_DEG + j) * CHUNK, CHUNK)],
                             ib, si)

        def iwait(j, ib, si):
            pltpu.make_async_copy(
                e_hbm.at[pl.ds((s * K_DEG + j) * CHUNK, CHUNK)], ib, si).wait()

        istart(0, ib0, si0)

        def body(i, carry):
            j0 = 2 * i
            j1 = 2 * i + 1
            istart(j1, ib1, si1)
            iwait(j0, ib0, si0)
            pltpu.sync_copy(ones_v, acc_sh.at[ib0], add=True)

            @pl.when(j1 + 1 < K_DEG)
            def _():
                istart(j1 + 1, ib0, si0)

            iwait(j1, ib1, si1)
            pltpu.sync_copy(ones_v, acc_sh.at[ib1], add=True)
            return carry

        lax.fori_loop(0, K_DEG // 2, body, 0)

    @pl.when(c == 0)
    def _():
        make_loop(src_hbm)

    @pl.when(c == 1)
    def _():
        make_loop(dst_hbm)

    plsc.subcore_barrier()

    @pl.when(c == 0)
    def _():
        pltpu.sync_copy(acc_sh.at[pl.ds(s * seg, seg)],
                        outs_hbm.at[pl.ds(s * seg, seg)])

    @pl.when(c == 1)
    def _():
        pltpu.sync_copy(acc_sh.at[pl.ds(s * seg, seg)],
                        outd_hbm.at[pl.ds(s * seg, seg)])


@functools.partial(
    pl.kernel,
    out_type=jax.ShapeDtypeStruct((NC, NPAD, D), jnp.float32),
    mesh=_MESH,
    scratch_types=[
        pltpu.VMEM((CHUNK,), jnp.int32),
        pltpu.VMEM((CHUNK,), jnp.int32),
        pltpu.VMEM((CHUNK,), jnp.int32),
        pltpu.VMEM((CHUNK,), jnp.int32),
        pltpu.VMEM((CHUNK, D), jnp.float32),
        pltpu.VMEM((CHUNK, D), jnp.float32),
        pltpu.VMEM_SHARED((NPAD, D), jnp.float32),
        pltpu.SemaphoreType.DMA,
        pltpu.SemaphoreType.DMA,
        pltpu.SemaphoreType.DMA,
        pltpu.SemaphoreType.DMA,
    ],
)
def _edge_agg_sc(m_hbm, src_hbm, dst_hbm, zrows_hbm, out_hbm,
                 sb0, sb1, db0, db1, rows0_v, rows1_v, acc_sh,
                 ss0, ss1, sg0, sg1):
    c = lax.axis_index("c")
    s = lax.axis_index("s")
    wid = s * NC + c

    pltpu.sync_copy(zrows_hbm.at[pl.ds(s * ROWS_PER_TILE, ROWS_PER_TILE)],
                    acc_sh.at[pl.ds(s * ROWS_PER_TILE, ROWS_PER_TILE)])
    plsc.subcore_barrier()

    base = wid * K_AGG

    def istart(j, ib, db, si):
        pltpu.async_copy(src_hbm.at[pl.ds((base + j) * CHUNK, CHUNK)], ib, si)
        pltpu.async_copy(dst_hbm.at[pl.ds((base + j) * CHUNK, CHUNK)], db, si)

    def iwait(j, ib, db, si):
        pltpu.make_async_copy(
            src_hbm.at[pl.ds((base + j) * CHUNK, CHUNK)], ib, si).wait()
        pltpu.make_async_copy(
            dst_hbm.at[pl.ds((base + j) * CHUNK, CHUNK)], db, si).wait()

    def gstart(ib, rows_ref, sg):
        pltpu.async_copy(m_hbm.at[ib], rows_ref, sg)

    def gwait(ib, rows_ref, sg):
        pltpu.make_async_copy(m_hbm.at[ib], rows_ref, sg).wait()

    istart(0, sb0, db0, ss0)
    iwait(0, sb0, db0, ss0)
    gstart(sb0, rows0_v, sg0)
    istart(1, sb1, db1, ss1)

    def body(i, carry):
        j0 = 2 * i
        j1 = 2 * i + 1
        iwait(j1, sb1, db1, ss1)
        gstart(sb1, rows1_v, sg1)
        gwait(sb0, rows0_v, sg0)
        pltpu.sync_copy(rows0_v, acc_sh.at[db0], add=True)

        @pl.when(j0 + 2 < K_AGG)
        def _():
            istart(j0 + 2, sb0, db0, ss0)
            iwait(j0 + 2, sb0, db0, ss0)
            gstart(sb0, rows0_v, sg0)

        @pl.when(j1 + 2 < K_AGG)
        def _():
            istart(j1 + 2, sb1, db1, ss1)

        gwait(sb1, rows1_v, sg1)
        pltpu.sync_copy(rows1_v, acc_sh.at[db1], add=True)
        return carry

    lax.fori_loop(0, K_AGG // 2, body, 0)
    plsc.subcore_barrier()

    pltpu.sync_copy(acc_sh.at[pl.ds(s * ROWS_PER_TILE, ROWS_PER_TILE)],
                    out_hbm.at[c, pl.ds(s * ROWS_PER_TILE, ROWS_PER_TILE)])


def _tc1_body(cnt_ref, x_ref, w1_ref, m1_ref, sout_ref, sin_ref):
    cnt = cnt_ref[...]
    sc = lax.rsqrt(jnp.maximum(cnt, 1.0))
    sout = sc[:, 0:1]
    sin = sc[:, 1:2]
    sout_ref[...] = sout
    sin_ref[...] = sin
    m1_ref[...] = jnp.dot(x_ref[...] * sout, w1_ref[...],
                          preferred_element_type=jnp.float32)


def _tc2_body(p_ref, sin_ref, b1_ref, sout_ref, w2_ref, m2_ref):
    agg = p_ref[0] + p_ref[1]
    h1 = jnp.maximum(agg * sin_ref[...] + b1_ref[...][None, :], 0.0)
    m2_ref[...] = jnp.dot(h1 * sout_ref[...], w2_ref[...],
                          preferred_element_type=jnp.float32)


def _tc3_body(p_ref, sin_ref, b2_ref, wc_ref, bc_ref, wf_ref, bf_ref,
              cat_ref, feat_ref):
    agg = p_ref[0] + p_ref[1]
    h2 = jnp.maximum(agg * sin_ref[...] + b2_ref[...][None, :], 0.0)
    cat_ref[...] = jnp.dot(h2, wc_ref[...],
                           preferred_element_type=jnp.float32) + bc_ref[...][None, :]
    feat_ref[...] = jnp.dot(h2, wf_ref[...],
                            preferred_element_type=jnp.float32) + bf_ref[...][None, :]


def kernel(x, edge_index, W1, b1, W2, b2, Wc, bc, Wf, bf):
    npad_e = EPAD - E
    pad_idx = N + (jnp.arange(npad_e, dtype=jnp.int32) % (NPAD - N))
    src1d = jnp.concatenate([edge_index[0], pad_idx])
    dst1d = jnp.concatenate([edge_index[1], pad_idx])
    xpad = jnp.pad(x, ((0, NPAD - N), (0, 0)))
    zeros_cnt = jnp.zeros((NPAD,), jnp.float32)
    zeros_rows = jnp.zeros((NPAD, D), jnp.float32)

    cnt_src, cnt_dst = _degrees_sc(src1d, dst1d, zeros_cnt)
    cnt_t = jnp.stack([cnt_src, cnt_dst], axis=1)

    m1, s_out, s_in = pl.pallas_call(
        _tc1_body,
        out_shape=(jax.ShapeDtypeStruct((NPAD, D), jnp.float32),
                   jax.ShapeDtypeStruct((NPAD, 1), jnp.float32),
                   jax.ShapeDtypeStruct((NPAD, 1), jnp.float32)),
    )(cnt_t, xpad, W1)

    p1 = _edge_agg_sc(m1, src1d, dst1d, zeros_rows)

    m2 = pl.pallas_call(
        _tc2_body,
        out_shape=jax.ShapeDtypeStruct((NPAD, D), jnp.float32),
    )(p1, s_in, b1, s_out, W2)

    p2 = _edge_agg_sc(m2, src1d, dst1d, zeros_rows)

    cat, feat = pl.pallas_call(
        _tc3_body,
        out_shape=(jax.ShapeDtypeStruct((NPAD, Wc.shape[1]), jnp.float32),
                   jax.ShapeDtypeStruct((NPAD, D), jnp.float32)),
    )(p2, s_in, b2, Wc, bc, Wf, bf)
    return (cat[:N], feat[:N])
